# Initial kernel scaffold; baseline (speedup 1.0000x reference)
import jax
import jax.numpy as jnp
from jax.experimental import pallas as pl


def kernel(x, targets, token_embedding_table):
    raise NotImplementedError("write your pallas kernel here")


# speedup vs baseline: 1.5580x; 1.5580x over previous
import functools

import jax
import jax.numpy as jnp
from jax import lax
from jax.experimental import pallas as pl
from jax.experimental.pallas import tpu as pltpu
from jax.experimental.pallas import tpu_sc as plsc

N_ROWS = 32768
D = 2048
NC, NS = 2, 16
NW = NC * NS
B_PER_W = N_ROWS // NW
CHUNK = 16
N_CHUNKS = B_PER_W // CHUNK


def _sc_gather(table, idx_flat):
    mesh = plsc.VectorSubcoreMesh(core_axis_name="c", subcore_axis_name="s")

    @functools.partial(
        pl.kernel,
        mesh=mesh,
        out_type=jax.ShapeDtypeStruct((N_ROWS, D), jnp.float32),
        scratch_types=[
            pltpu.VMEM((B_PER_W,), jnp.int32),
            pltpu.VMEM((CHUNK, D), jnp.float32),
            pltpu.SemaphoreType.DMA,
        ],
    )
    def gather_kernel(table_hbm, idx_hbm, out_hbm, idx_v, rows_v, sem):
        wid = lax.axis_index("s") * NC + lax.axis_index("c")
        base = wid * B_PER_W
        pltpu.sync_copy(idx_hbm.at[pl.ds(base, B_PER_W)], idx_v)

        def body(i, carry):
            off = i * CHUNK
            pltpu.async_copy(
                table_hbm.at[idx_v.at[pl.ds(off, CHUNK)]], rows_v, sem
            ).wait()
            pltpu.sync_copy(rows_v, out_hbm.at[pl.ds(base + off, CHUNK)])
            return carry

        lax.fori_loop(0, N_CHUNKS, body, 0)

    return gather_kernel(table, idx_flat)


ROWS_BLK = 256
N_BLKS = N_ROWS // ROWS_BLK


def _tc_loss_kernel(tgt_ref, logits_ref, acc_ref):
    i = pl.program_id(0)
    blk = logits_ref[...]
    m = jnp.max(blk, axis=1, keepdims=True)
    lse = jnp.log(jnp.sum(jnp.exp(blk - m), axis=1, keepdims=True)) + m
    t = tgt_ref[0, 0, :]
    cols = lax.broadcasted_iota(jnp.int32, (ROWS_BLK, D), 1)
    picked = jnp.sum(
        jnp.where(cols == t[:, None], blk, 0.0), axis=1, keepdims=True
    )
    part = jnp.sum(lse - picked)

    @pl.when(i == 0)
    def _():
        acc_ref[0, 0] = 0.0

    acc_ref[0, 0] += part

    @pl.when(i == N_BLKS - 1)
    def _():
        acc_ref[0, 0] = acc_ref[0, 0] / N_ROWS


def _tc_loss(logits, targets_flat):
    tgt3 = targets_flat.reshape(N_BLKS, 1, ROWS_BLK)
    acc = pl.pallas_call(
        _tc_loss_kernel,
        grid=(N_BLKS,),
        in_specs=[
            pl.BlockSpec((1, 1, ROWS_BLK), lambda i: (i, 0, 0)),
            pl.BlockSpec((ROWS_BLK, D), lambda i: (i, 0)),
        ],
        out_specs=pl.BlockSpec(
            (1, 1), lambda i: (0, 0), memory_space=pltpu.SMEM
        ),
        out_shape=jax.ShapeDtypeStruct((1, 1), jnp.float32),
    )(tgt3, logits)
    return acc[0, 0]


def kernel(x, targets, token_embedding_table):
    idx_flat = x.reshape(N_ROWS)
    tgt_flat = targets.reshape(N_ROWS)
    logits = _sc_gather(token_embedding_table, idx_flat)
    loss = _tc_loss(logits, tgt_flat)
    return (logits, loss)


# speedup vs baseline: 1.7518x; 1.1244x over previous
import functools

import jax
import jax.numpy as jnp
from jax import lax
from jax.experimental import pallas as pl
from jax.experimental.pallas import tpu as pltpu
from jax.experimental.pallas import tpu_sc as plsc

N_ROWS = 32768
D = 2048
NC, NS = 2, 16
NW = NC * NS
B_PER_W = N_ROWS // NW
CHUNK = 16
N_CHUNKS = B_PER_W // CHUNK


def _sc_gather(table, idx_flat):
    mesh = plsc.VectorSubcoreMesh(core_axis_name="c", subcore_axis_name="s")

    @functools.partial(
        pl.kernel,
        mesh=mesh,
        out_type=jax.ShapeDtypeStruct((N_ROWS, D), jnp.float32),
        scratch_types=[
            pltpu.VMEM((B_PER_W,), jnp.int32),
            pltpu.VMEM((CHUNK, D), jnp.float32),
            pltpu.VMEM((CHUNK, D), jnp.float32),
            pltpu.SemaphoreType.DMA,
            pltpu.SemaphoreType.DMA,
        ],
    )
    def gather_kernel(table_hbm, idx_hbm, out_hbm, idx_v, buf0, buf1, sem0, sem1):
        wid = lax.axis_index("s") * NC + lax.axis_index("c")
        base = wid * B_PER_W
        pltpu.sync_copy(idx_hbm.at[pl.ds(base, B_PER_W)], idx_v)
        bufs = (buf0, buf1)
        sems = (sem0, sem1)

        def gather_chunk(i, b):
            pltpu.async_copy(
                table_hbm.at[idx_v.at[pl.ds(i * CHUNK, CHUNK)]], bufs[b], sems[b]
            )

        for b in range(2):
            gather_chunk(b, b)

        def body(j, carry):
            for b in range(2):
                i = j * 2 + b
                pltpu.make_async_copy(
                    table_hbm.at[idx_v.at[pl.ds(i * CHUNK, CHUNK)]],
                    bufs[b],
                    sems[b],
                ).wait()
                pltpu.sync_copy(bufs[b], out_hbm.at[pl.ds(base + i * CHUNK, CHUNK)])

                @pl.when(i + 2 < N_CHUNKS)
                def _():
                    gather_chunk(i + 2, b)

            return carry

        lax.fori_loop(0, N_CHUNKS // 2, body, 0)

    return gather_kernel(table, idx_flat)


ROWS_BLK = 256
N_BLKS = N_ROWS // ROWS_BLK


def _tc_loss_kernel(tgt_ref, logits_ref, acc_ref):
    i = pl.program_id(0)
    blk = logits_ref[...]
    m = jnp.max(blk, axis=1, keepdims=True)
    lse = jnp.log(jnp.sum(jnp.exp(blk - m), axis=1, keepdims=True)) + m
    t = tgt_ref[0, 0, :]
    cols = lax.broadcasted_iota(jnp.int32, (ROWS_BLK, D), 1)
    picked = jnp.sum(
        jnp.where(cols == t[:, None], blk, 0.0), axis=1, keepdims=True
    )
    part = jnp.sum(lse - picked)

    @pl.when(i == 0)
    def _():
        acc_ref[0, 0] = 0.0

    acc_ref[0, 0] += part

    @pl.when(i == N_BLKS - 1)
    def _():
        acc_ref[0, 0] = acc_ref[0, 0] / N_ROWS


def _tc_loss(logits, targets_flat):
    tgt3 = targets_flat.reshape(N_BLKS, 1, ROWS_BLK)
    acc = pl.pallas_call(
        _tc_loss_kernel,
        grid=(N_BLKS,),
        in_specs=[
            pl.BlockSpec((1, 1, ROWS_BLK), lambda i: (i, 0, 0)),
            pl.BlockSpec((ROWS_BLK, D), lambda i: (i, 0)),
        ],
        out_specs=pl.BlockSpec(
            (1, 1), lambda i: (0, 0), memory_space=pltpu.SMEM
        ),
        out_shape=jax.ShapeDtypeStruct((1, 1), jnp.float32),
    )(tgt3, logits)
    return acc[0, 0]


def kernel(x, targets, token_embedding_table):
    idx_flat = x.reshape(N_ROWS)
    tgt_flat = targets.reshape(N_ROWS)
    logits = _sc_gather(token_embedding_table, idx_flat)
    loss = _tc_loss(logits, tgt_flat)
    return (logits, loss)
